# baseline (speedup 1.0000x reference)
import functools

import jax
import jax.numpy as jnp
from jax import lax
from jax.experimental import pallas as pl
from jax.experimental.pallas import tpu as pltpu
from jax.experimental.pallas import tpu_sc as plsc

N = 10000
N_PAD = 10240
E = 320000
CH = 128
NC = 2
NS = 16
NW = NC * NS
ROWS_PER_TILE = N_PAD // NS
K = 8
CPW = 80
E_PAD = NW * CPW * CH
NITER = CPW // K

@functools.cache
def _make_sc_aggregate(F):
    @functools.partial(
        pl.kernel,
        out_type=jax.ShapeDtypeStruct((NC, N_PAD, F), jnp.float32),
        mesh=plsc.VectorSubcoreMesh(core_axis_name="c", subcore_axis_name="s"),
        compiler_params=pltpu.CompilerParams(use_tc_tiling_on_sc=False),
        scratch_types=[
            pltpu.VMEM((CPW, CH), jnp.int32),
            pltpu.VMEM((CPW, CH), jnp.int32),
            pltpu.VMEM((K, CH, F), jnp.float32),
            pltpu.VMEM_SHARED((N_PAD, F), jnp.float32),
            [pltpu.SemaphoreType.DMA] * K,
            pltpu.SemaphoreType.DMA,
        ],
    )
    def agg(src_hbm, dst_hbm, s_hbm, out_hbm, sidx_v, didx_v, bufs_v, acc_sh,
            gsems, ssem):
        cid = lax.axis_index("c")
        sid = lax.axis_index("s")
        wid = sid * NC + cid
        base = sid * ROWS_PER_TILE
        pltpu.sync_copy(s_hbm.at[pl.ds(base, ROWS_PER_TILE)],
                        acc_sh.at[pl.ds(base, ROWS_PER_TILE)])
        pltpu.sync_copy(src_hbm.at[wid], sidx_v)
        pltpu.sync_copy(dst_hbm.at[wid], didx_v)
        plsc.subcore_barrier()

        def body(p, carry):
            i0 = p * K
            gathers = [
                pltpu.async_copy(s_hbm.at[sidx_v.at[i0 + k]], bufs_v.at[k],
                                 gsems[k])
                for k in range(K)
            ]
            scatters = []
            for k in range(K):
                gathers[k].wait()
                scatters.append(
                    pltpu.async_copy(bufs_v.at[k], acc_sh.at[didx_v.at[i0 + k]],
                                     ssem, add=True))
            for k in range(K):
                scatters[k].wait()
            return carry

        lax.fori_loop(0, NITER, body, 0)
        plsc.subcore_barrier()
        pltpu.sync_copy(acc_sh.at[pl.ds(base, ROWS_PER_TILE)],
                        out_hbm.at[cid, pl.ds(base, ROWS_PER_TILE)])

    return agg


@functools.cache
def _make_sc_degree():

    @functools.partial(
        pl.kernel,
        out_type=jax.ShapeDtypeStruct((NC, N_PAD, 8), jnp.float32),
        mesh=plsc.VectorSubcoreMesh(core_axis_name="c", subcore_axis_name="s"),
        compiler_params=pltpu.CompilerParams(use_tc_tiling_on_sc=False),
        scratch_types=[
            pltpu.VMEM((CPW, CH), jnp.int32),
            pltpu.VMEM((CH, 8), jnp.float32),
            pltpu.VMEM_SHARED((N_PAD, 8), jnp.float32),
            pltpu.SemaphoreType.DMA,
        ],
    )
    def _sc_degree(dst_hbm, ones_hbm, out_hbm, didx_v, ones_v, acc_sh, ssem):
        cid = lax.axis_index("c")
        sid = lax.axis_index("s")
        wid = sid * NC + cid
        base = sid * ROWS_PER_TILE
        pltpu.sync_copy(ones_hbm, acc_sh.at[pl.ds(base, ROWS_PER_TILE)])
        pltpu.sync_copy(ones_hbm.at[pl.ds(0, CH)], ones_v)
        pltpu.sync_copy(dst_hbm.at[wid], didx_v)
        plsc.subcore_barrier()

        def body(p, carry):
            i0 = p * K
            scatters = [
                pltpu.async_copy(ones_v, acc_sh.at[didx_v.at[i0 + k]], ssem,
                                 add=True)
                for k in range(K)
            ]
            for k in range(K):
                scatters[k].wait()
            return carry

        lax.fori_loop(0, NITER, body, 0)
        plsc.subcore_barrier()
        pltpu.sync_copy(acc_sh.at[pl.ds(base, ROWS_PER_TILE)],
                        out_hbm.at[cid, pl.ds(base, ROWS_PER_TILE)])

    return _sc_degree


_R = 2048
_GRID = N_PAD // _R


def _tc_pre_body(parts_ref, x_ref, w_ref, s_ref, dinv_ref):
    deg = parts_ref[0, :, 0:1] + parts_ref[1, :, 0:1] - 1.0
    dinv = lax.rsqrt(jnp.maximum(deg, 1.0))
    dinv_ref[...] = dinv
    s_ref[...] = jnp.dot(x_ref[...], w_ref[...],
                         preferred_element_type=jnp.float32) * dinv


def _tc_pre(deg_parts, x_pad, w1):
    return pl.pallas_call(
        _tc_pre_body,
        grid=(_GRID,),
        in_specs=[
            pl.BlockSpec((NC, _R, 8), lambda i: (0, i, 0)),
            pl.BlockSpec((_R, 128), lambda i: (i, 0)),
            pl.BlockSpec((128, 64), lambda i: (0, 0)),
        ],
        out_specs=[
            pl.BlockSpec((_R, 64), lambda i: (i, 0)),
            pl.BlockSpec((_R, 1), lambda i: (i, 0)),
        ],
        out_shape=[
            jax.ShapeDtypeStruct((N_PAD, 64), jnp.float32),
            jax.ShapeDtypeStruct((N_PAD, 1), jnp.float32),
        ],
    )(deg_parts, x_pad, w1)


def _tc_mid_body(parts_ref, s1_ref, dinv_ref, b1_ref, w2_ref, s2_ref):
    agg = parts_ref[0] + parts_ref[1] - s1_ref[...]
    h = jnp.maximum(agg * dinv_ref[...] + b1_ref[...], 0.0)
    s2_ref[...] = jnp.dot(h, w2_ref[...],
                          preferred_element_type=jnp.float32) * dinv_ref[...]


def _tc_mid(parts1, s1, dinv, b1r, w2p):
    return pl.pallas_call(
        _tc_mid_body,
        grid=(_GRID,),
        in_specs=[
            pl.BlockSpec((NC, _R, 64), lambda i: (0, i, 0)),
            pl.BlockSpec((_R, 64), lambda i: (i, 0)),
            pl.BlockSpec((_R, 1), lambda i: (i, 0)),
            pl.BlockSpec((1, 64), lambda i: (0, 0)),
            pl.BlockSpec((64, 8), lambda i: (0, 0)),
        ],
        out_specs=pl.BlockSpec((_R, 8), lambda i: (i, 0)),
        out_shape=jax.ShapeDtypeStruct((N_PAD, 8), jnp.float32),
    )(parts1, s1, dinv, b1r, w2p)


def _tc_post_body(parts_ref, s2_ref, dinv_ref, b2_ref, o_ref):
    agg = parts_ref[0] + parts_ref[1] - s2_ref[...]
    o_ref[...] = agg * dinv_ref[...] + b2_ref[...]


def _tc_post(parts2, s2, dinv, b2r):
    return pl.pallas_call(
        _tc_post_body,
        grid=(_GRID,),
        in_specs=[
            pl.BlockSpec((NC, _R, 8), lambda i: (0, i, 0)),
            pl.BlockSpec((_R, 8), lambda i: (i, 0)),
            pl.BlockSpec((_R, 1), lambda i: (i, 0)),
            pl.BlockSpec((1, 8), lambda i: (0, 0)),
        ],
        out_specs=pl.BlockSpec((_R, 8), lambda i: (i, 0)),
        out_shape=jax.ShapeDtypeStruct((N_PAD, 8), jnp.float32),
    )(parts2, s2, dinv, b2r)


def kernel(x, edge_index, W1, b1, W2, b2):
    ei = edge_index.astype(jnp.int32)
    pad_dst = N + (jnp.arange(E_PAD - E, dtype=jnp.int32) % (N_PAD - N))
    src_r = jnp.concatenate(
        [ei[0], jnp.zeros((E_PAD - E,), jnp.int32)]).reshape(NW, CPW, CH)
    dst_r = jnp.concatenate([ei[1], pad_dst]).reshape(NW, CPW, CH)
    x_pad = jnp.pad(x, ((0, N_PAD - N), (0, 0)))
    w2p = jnp.pad(W2, ((0, 0), (0, 1)))
    b1r = b1.reshape(1, 64)
    b2r = jnp.pad(b2, (0, 1)).reshape(1, 8)
    ones = jnp.ones((ROWS_PER_TILE, 8), jnp.float32)

    deg_parts = _make_sc_degree()(dst_r, ones)
    s1, dinv = _tc_pre(deg_parts, x_pad, W1)
    parts1 = _make_sc_aggregate(64)(src_r, dst_r, s1)
    s2 = _tc_mid(parts1, s1, dinv, b1r, w2p)
    parts2 = _make_sc_aggregate(8)(src_r, dst_r, s2)
    outp = _tc_post(parts2, s2, dinv, b2r)
    return outp[:N, :7]


# speedup vs baseline: 1.8214x; 1.8214x over previous
import functools

import jax
import jax.numpy as jnp
from jax import lax
from jax.experimental import pallas as pl
from jax.experimental.pallas import tpu as pltpu
from jax.experimental.pallas import tpu_sc as plsc

N = 10000
N_PAD = 10240
E = 320000
CH = 128
NC = 2
NS = 16
NW = NC * NS
ROWS_PER_TILE = N_PAD // NS
K = 8
CPW = 80
E_PAD = NW * CPW * CH
NITER = CPW // K

@functools.cache
def _make_sc_aggregate(F, KP):
    NIT = CPW // KP

    @functools.partial(
        pl.kernel,
        out_type=jax.ShapeDtypeStruct((NC, N_PAD, F), jnp.float32),
        mesh=plsc.VectorSubcoreMesh(core_axis_name="c", subcore_axis_name="s"),
        compiler_params=pltpu.CompilerParams(use_tc_tiling_on_sc=False),
        scratch_types=[
            pltpu.VMEM((CPW, CH), jnp.int32),
            pltpu.VMEM((CPW, CH), jnp.int32),
            pltpu.VMEM((KP, CH, F), jnp.float32),
            pltpu.VMEM_SHARED((N_PAD, F), jnp.float32),
            pltpu.VMEM_SHARED((N_PAD, F), jnp.float32),
            [pltpu.SemaphoreType.DMA] * KP,
            pltpu.SemaphoreType.DMA,
        ],
    )
    def agg(src_hbm, dst_hbm, s_hbm, out_hbm, sidx_v, didx_v, bufs_v, acc_sh,
            s_sh, gsems, ssem):
        cid = lax.axis_index("c")
        sid = lax.axis_index("s")
        wid = sid * NC + cid
        base = sid * ROWS_PER_TILE
        pltpu.sync_copy(s_hbm.at[pl.ds(base, ROWS_PER_TILE)],
                        s_sh.at[pl.ds(base, ROWS_PER_TILE)])
        pltpu.sync_copy(s_hbm.at[pl.ds(base, ROWS_PER_TILE)],
                        acc_sh.at[pl.ds(base, ROWS_PER_TILE)])
        pltpu.sync_copy(src_hbm.at[wid], sidx_v)
        pltpu.sync_copy(dst_hbm.at[wid], didx_v)
        plsc.subcore_barrier()

        def body(p, carry):
            i0 = p * KP
            gathers = [
                pltpu.async_copy(s_sh.at[sidx_v.at[i0 + k]], bufs_v.at[k],
                                 gsems[k])
                for k in range(KP)
            ]
            scatters = []
            for k in range(KP):
                gathers[k].wait()
                scatters.append(
                    pltpu.async_copy(bufs_v.at[k], acc_sh.at[didx_v.at[i0 + k]],
                                     ssem, add=True))
            for k in range(KP):
                scatters[k].wait()
            return carry

        lax.fori_loop(0, NIT, body, 0)
        plsc.subcore_barrier()
        pltpu.sync_copy(acc_sh.at[pl.ds(base, ROWS_PER_TILE)],
                        out_hbm.at[cid, pl.ds(base, ROWS_PER_TILE)])

    return agg


@functools.cache
def _make_sc_degree():

    @functools.partial(
        pl.kernel,
        out_type=jax.ShapeDtypeStruct((NC, N_PAD, 8), jnp.float32),
        mesh=plsc.VectorSubcoreMesh(core_axis_name="c", subcore_axis_name="s"),
        compiler_params=pltpu.CompilerParams(use_tc_tiling_on_sc=False),
        scratch_types=[
            pltpu.VMEM((CPW, CH), jnp.int32),
            pltpu.VMEM((CH, 8), jnp.float32),
            pltpu.VMEM_SHARED((N_PAD, 8), jnp.float32),
            pltpu.SemaphoreType.DMA,
        ],
    )
    def _sc_degree(dst_hbm, ones_hbm, out_hbm, didx_v, ones_v, acc_sh, ssem):
        cid = lax.axis_index("c")
        sid = lax.axis_index("s")
        wid = sid * NC + cid
        base = sid * ROWS_PER_TILE
        pltpu.sync_copy(ones_hbm, acc_sh.at[pl.ds(base, ROWS_PER_TILE)])
        pltpu.sync_copy(ones_hbm.at[pl.ds(0, CH)], ones_v)
        pltpu.sync_copy(dst_hbm.at[wid], didx_v)
        plsc.subcore_barrier()

        def body(p, carry):
            i0 = p * K
            scatters = [
                pltpu.async_copy(ones_v, acc_sh.at[didx_v.at[i0 + k]], ssem,
                                 add=True)
                for k in range(K)
            ]
            for k in range(K):
                scatters[k].wait()
            return carry

        lax.fori_loop(0, NITER, body, 0)
        plsc.subcore_barrier()
        pltpu.sync_copy(acc_sh.at[pl.ds(base, ROWS_PER_TILE)],
                        out_hbm.at[cid, pl.ds(base, ROWS_PER_TILE)])

    return _sc_degree


_R = 2048
_GRID = N_PAD // _R


def _tc_pre_body(parts_ref, x_ref, w_ref, s_ref, dinv_ref):
    deg = parts_ref[0, :, 0:1] + parts_ref[1, :, 0:1] - 1.0
    dinv = lax.rsqrt(jnp.maximum(deg, 1.0))
    dinv_ref[...] = dinv
    s_ref[...] = jnp.dot(x_ref[...], w_ref[...],
                         preferred_element_type=jnp.float32) * dinv


def _tc_pre(deg_parts, x_pad, w1):
    return pl.pallas_call(
        _tc_pre_body,
        grid=(_GRID,),
        in_specs=[
            pl.BlockSpec((NC, _R, 8), lambda i: (0, i, 0)),
            pl.BlockSpec((_R, 128), lambda i: (i, 0)),
            pl.BlockSpec((128, 64), lambda i: (0, 0)),
        ],
        out_specs=[
            pl.BlockSpec((_R, 64), lambda i: (i, 0)),
            pl.BlockSpec((_R, 1), lambda i: (i, 0)),
        ],
        out_shape=[
            jax.ShapeDtypeStruct((N_PAD, 64), jnp.float32),
            jax.ShapeDtypeStruct((N_PAD, 1), jnp.float32),
        ],
    )(deg_parts, x_pad, w1)


def _tc_mid_body(parts_ref, s1_ref, dinv_ref, b1_ref, w2_ref, s2_ref):
    agg = parts_ref[0] + parts_ref[1] - s1_ref[...]
    h = jnp.maximum(agg * dinv_ref[...] + b1_ref[...], 0.0)
    s2_ref[...] = jnp.dot(h, w2_ref[...],
                          preferred_element_type=jnp.float32) * dinv_ref[...]


def _tc_mid(parts1, s1, dinv, b1r, w2p):
    return pl.pallas_call(
        _tc_mid_body,
        grid=(_GRID,),
        in_specs=[
            pl.BlockSpec((NC, _R, 64), lambda i: (0, i, 0)),
            pl.BlockSpec((_R, 64), lambda i: (i, 0)),
            pl.BlockSpec((_R, 1), lambda i: (i, 0)),
            pl.BlockSpec((1, 64), lambda i: (0, 0)),
            pl.BlockSpec((64, 8), lambda i: (0, 0)),
        ],
        out_specs=pl.BlockSpec((_R, 8), lambda i: (i, 0)),
        out_shape=jax.ShapeDtypeStruct((N_PAD, 8), jnp.float32),
    )(parts1, s1, dinv, b1r, w2p)


def _tc_post_body(parts_ref, s2_ref, dinv_ref, b2_ref, o_ref):
    agg = parts_ref[0] + parts_ref[1] - s2_ref[...]
    o_ref[...] = agg * dinv_ref[...] + b2_ref[...]


def _tc_post(parts2, s2, dinv, b2r):
    return pl.pallas_call(
        _tc_post_body,
        grid=(_GRID,),
        in_specs=[
            pl.BlockSpec((NC, _R, 8), lambda i: (0, i, 0)),
            pl.BlockSpec((_R, 8), lambda i: (i, 0)),
            pl.BlockSpec((_R, 1), lambda i: (i, 0)),
            pl.BlockSpec((1, 8), lambda i: (0, 0)),
        ],
        out_specs=pl.BlockSpec((_R, 8), lambda i: (i, 0)),
        out_shape=jax.ShapeDtypeStruct((N_PAD, 8), jnp.float32),
    )(parts2, s2, dinv, b2r)


def kernel(x, edge_index, W1, b1, W2, b2):
    ei = edge_index.astype(jnp.int32)
    pad_dst = N + (jnp.arange(E_PAD - E, dtype=jnp.int32) % (N_PAD - N))
    src_r = jnp.concatenate(
        [ei[0], jnp.zeros((E_PAD - E,), jnp.int32)]).reshape(NW, CPW, CH)
    dst_r = jnp.concatenate([ei[1], pad_dst]).reshape(NW, CPW, CH)
    x_pad = jnp.pad(x, ((0, N_PAD - N), (0, 0)))
    w2p = jnp.pad(W2, ((0, 0), (0, 1)))
    b1r = b1.reshape(1, 64)
    b2r = jnp.pad(b2, (0, 1)).reshape(1, 8)
    ones = jnp.ones((ROWS_PER_TILE, 8), jnp.float32)

    deg_parts = _make_sc_degree()(dst_r, ones)
    s1, dinv = _tc_pre(deg_parts, x_pad, W1)
    parts1 = _make_sc_aggregate(64, 2)(src_r, dst_r, s1)
    s2 = _tc_mid(parts1, s1, dinv, b1r, w2p)
    parts2 = _make_sc_aggregate(8, 8)(src_r, dst_r, s2)
    outp = _tc_post(parts2, s2, dinv, b2r)
    return outp[:N, :7]


# speedup vs baseline: 1.9571x; 1.0745x over previous
import functools

import jax
import jax.numpy as jnp
from jax import lax
from jax.experimental import pallas as pl
from jax.experimental.pallas import tpu as pltpu
from jax.experimental.pallas import tpu_sc as plsc

N = 10000
N_PAD = 10240
E = 320000
CH = 128
NC = 2
NS = 16
NW = NC * NS
ROWS_PER_TILE = N_PAD // NS
K = 8
CPW = 80
E_PAD = NW * CPW * CH
NITER = CPW // K

@functools.cache
def _make_sc_aggregate(F, KP):
    NST = 2 if KP * CH * F * NS + 2 * CPW * CH * NS > 851968 else 1
    CPS = CPW // NST
    NIT = CPS // KP

    @functools.partial(
        pl.kernel,
        out_type=jax.ShapeDtypeStruct((NC, N_PAD, F), jnp.float32),
        mesh=plsc.VectorSubcoreMesh(core_axis_name="c", subcore_axis_name="s"),
        compiler_params=pltpu.CompilerParams(use_tc_tiling_on_sc=False),
        scratch_types=[
            pltpu.VMEM((CPS, CH), jnp.int32),
            pltpu.VMEM((CPS, CH), jnp.int32),
            pltpu.VMEM((KP, CH, F), jnp.float32),
            pltpu.VMEM_SHARED((N_PAD, F), jnp.float32),
            pltpu.VMEM_SHARED((N_PAD, F), jnp.float32),
            [pltpu.SemaphoreType.DMA] * KP,
            pltpu.SemaphoreType.DMA,
        ],
    )
    def agg(src_hbm, dst_hbm, s_hbm, out_hbm, sidx_v, didx_v, bufs_v, acc_sh,
            s_sh, gsems, ssem):
        cid = lax.axis_index("c")
        sid = lax.axis_index("s")
        wid = sid * NC + cid
        base = sid * ROWS_PER_TILE
        pltpu.sync_copy(s_hbm.at[pl.ds(base, ROWS_PER_TILE)],
                        s_sh.at[pl.ds(base, ROWS_PER_TILE)])
        pltpu.sync_copy(s_hbm.at[pl.ds(base, ROWS_PER_TILE)],
                        acc_sh.at[pl.ds(base, ROWS_PER_TILE)])
        plsc.subcore_barrier()

        def body(p, carry):
            i0 = p * KP
            gathers = [
                pltpu.async_copy(s_sh.at[sidx_v.at[i0 + k]], bufs_v.at[k],
                                 gsems[k])
                for k in range(KP)
            ]
            scatters = []
            for k in range(KP):
                gathers[k].wait()
                scatters.append(
                    pltpu.async_copy(bufs_v.at[k], acc_sh.at[didx_v.at[i0 + k]],
                                     ssem, add=True))
            for k in range(KP):
                scatters[k].wait()
            return carry

        for b in range(NST):
            pltpu.sync_copy(src_hbm.at[wid, pl.ds(b * CPS, CPS)], sidx_v)
            pltpu.sync_copy(dst_hbm.at[wid, pl.ds(b * CPS, CPS)], didx_v)
            lax.fori_loop(0, NIT, body, 0)

        plsc.subcore_barrier()
        pltpu.sync_copy(acc_sh.at[pl.ds(base, ROWS_PER_TILE)],
                        out_hbm.at[cid, pl.ds(base, ROWS_PER_TILE)])

    return agg


@functools.cache
def _make_sc_degree():

    @functools.partial(
        pl.kernel,
        out_type=jax.ShapeDtypeStruct((NC, N_PAD, 8), jnp.float32),
        mesh=plsc.VectorSubcoreMesh(core_axis_name="c", subcore_axis_name="s"),
        compiler_params=pltpu.CompilerParams(use_tc_tiling_on_sc=False),
        scratch_types=[
            pltpu.VMEM((CPW, CH), jnp.int32),
            pltpu.VMEM((CH, 8), jnp.float32),
            pltpu.VMEM_SHARED((N_PAD, 8), jnp.float32),
            pltpu.SemaphoreType.DMA,
        ],
    )
    def _sc_degree(dst_hbm, ones_hbm, out_hbm, didx_v, ones_v, acc_sh, ssem):
        cid = lax.axis_index("c")
        sid = lax.axis_index("s")
        wid = sid * NC + cid
        base = sid * ROWS_PER_TILE
        pltpu.sync_copy(ones_hbm, acc_sh.at[pl.ds(base, ROWS_PER_TILE)])
        pltpu.sync_copy(ones_hbm.at[pl.ds(0, CH)], ones_v)
        pltpu.sync_copy(dst_hbm.at[wid], didx_v)
        plsc.subcore_barrier()

        def body(p, carry):
            i0 = p * K
            scatters = [
                pltpu.async_copy(ones_v, acc_sh.at[didx_v.at[i0 + k]], ssem,
                                 add=True)
                for k in range(K)
            ]
            for k in range(K):
                scatters[k].wait()
            return carry

        lax.fori_loop(0, NITER, body, 0)
        plsc.subcore_barrier()
        pltpu.sync_copy(acc_sh.at[pl.ds(base, ROWS_PER_TILE)],
                        out_hbm.at[cid, pl.ds(base, ROWS_PER_TILE)])

    return _sc_degree


_R = 2048
_GRID = N_PAD // _R


def _tc_pre_body(parts_ref, x_ref, w_ref, s_ref, dinv_ref):
    deg = parts_ref[0, :, 0:1] + parts_ref[1, :, 0:1] - 1.0
    dinv = lax.rsqrt(jnp.maximum(deg, 1.0))
    dinv_ref[...] = dinv
    s_ref[...] = jnp.dot(x_ref[...], w_ref[...],
                         preferred_element_type=jnp.float32) * dinv


def _tc_pre(deg_parts, x_pad, w1):
    return pl.pallas_call(
        _tc_pre_body,
        grid=(_GRID,),
        in_specs=[
            pl.BlockSpec((NC, _R, 8), lambda i: (0, i, 0)),
            pl.BlockSpec((_R, 128), lambda i: (i, 0)),
            pl.BlockSpec((128, 64), lambda i: (0, 0)),
        ],
        out_specs=[
            pl.BlockSpec((_R, 64), lambda i: (i, 0)),
            pl.BlockSpec((_R, 1), lambda i: (i, 0)),
        ],
        out_shape=[
            jax.ShapeDtypeStruct((N_PAD, 64), jnp.float32),
            jax.ShapeDtypeStruct((N_PAD, 1), jnp.float32),
        ],
    )(deg_parts, x_pad, w1)


def _tc_mid_body(parts_ref, s1_ref, dinv_ref, b1_ref, w2_ref, s2_ref):
    agg = parts_ref[0] + parts_ref[1] - s1_ref[...]
    h = jnp.maximum(agg * dinv_ref[...] + b1_ref[...], 0.0)
    s2_ref[...] = jnp.dot(h, w2_ref[...],
                          preferred_element_type=jnp.float32) * dinv_ref[...]


def _tc_mid(parts1, s1, dinv, b1r, w2p):
    return pl.pallas_call(
        _tc_mid_body,
        grid=(_GRID,),
        in_specs=[
            pl.BlockSpec((NC, _R, 64), lambda i: (0, i, 0)),
            pl.BlockSpec((_R, 64), lambda i: (i, 0)),
            pl.BlockSpec((_R, 1), lambda i: (i, 0)),
            pl.BlockSpec((1, 64), lambda i: (0, 0)),
            pl.BlockSpec((64, 8), lambda i: (0, 0)),
        ],
        out_specs=pl.BlockSpec((_R, 8), lambda i: (i, 0)),
        out_shape=jax.ShapeDtypeStruct((N_PAD, 8), jnp.float32),
    )(parts1, s1, dinv, b1r, w2p)


def _tc_post_body(parts_ref, s2_ref, dinv_ref, b2_ref, o_ref):
    agg = parts_ref[0] + parts_ref[1] - s2_ref[...]
    o_ref[...] = agg * dinv_ref[...] + b2_ref[...]


def _tc_post(parts2, s2, dinv, b2r):
    return pl.pallas_call(
        _tc_post_body,
        grid=(_GRID,),
        in_specs=[
            pl.BlockSpec((NC, _R, 8), lambda i: (0, i, 0)),
            pl.BlockSpec((_R, 8), lambda i: (i, 0)),
            pl.BlockSpec((_R, 1), lambda i: (i, 0)),
            pl.BlockSpec((1, 8), lambda i: (0, 0)),
        ],
        out_specs=pl.BlockSpec((_R, 8), lambda i: (i, 0)),
        out_shape=jax.ShapeDtypeStruct((N_PAD, 8), jnp.float32),
    )(parts2, s2, dinv, b2r)


def kernel(x, edge_index, W1, b1, W2, b2):
    ei = edge_index.astype(jnp.int32)
    pad_dst = N + (jnp.arange(E_PAD - E, dtype=jnp.int32) % (N_PAD - N))
    src_r = jnp.concatenate(
        [ei[0], jnp.zeros((E_PAD - E,), jnp.int32)]).reshape(NW, CPW, CH)
    dst_r = jnp.concatenate([ei[1], pad_dst]).reshape(NW, CPW, CH)
    x_pad = jnp.pad(x, ((0, N_PAD - N), (0, 0)))
    w2p = jnp.pad(W2, ((0, 0), (0, 1)))
    b1r = b1.reshape(1, 64)
    b2r = jnp.pad(b2, (0, 1)).reshape(1, 8)
    ones = jnp.ones((ROWS_PER_TILE, 8), jnp.float32)

    deg_parts = _make_sc_degree()(dst_r, ones)
    s1, dinv = _tc_pre(deg_parts, x_pad, W1)
    parts1 = _make_sc_aggregate(64, 4)(src_r, dst_r, s1)
    s2 = _tc_mid(parts1, s1, dinv, b1r, w2p)
    parts2 = _make_sc_aggregate(8, 8)(src_r, dst_r, s2)
    outp = _tc_post(parts2, s2, dinv, b2r)
    return outp[:N, :7]


# speedup vs baseline: 1.9659x; 1.0045x over previous
import functools

import jax
import jax.numpy as jnp
from jax import lax
from jax.experimental import pallas as pl
from jax.experimental.pallas import tpu as pltpu
from jax.experimental.pallas import tpu_sc as plsc

N = 10000
N_PAD = 10240
E = 320000
CH = 128
NC = 2
NS = 16
NW = NC * NS
ROWS_PER_TILE = N_PAD // NS
K = 8
CPW = 80
E_PAD = NW * CPW * CH
NITER = CPW // K

@functools.cache
def _make_sc_aggregate(F, KP):
    NST = 2 if KP * CH * F * NS + 2 * CPW * CH * NS > 851968 else 1
    CPS = CPW // NST
    NIT = CPS // KP

    @functools.partial(
        pl.kernel,
        out_type=jax.ShapeDtypeStruct((NC, N_PAD, F), jnp.float32),
        mesh=plsc.VectorSubcoreMesh(core_axis_name="c", subcore_axis_name="s"),
        compiler_params=pltpu.CompilerParams(use_tc_tiling_on_sc=False),
        scratch_types=[
            pltpu.VMEM((CPS, CH), jnp.int32),
            pltpu.VMEM((CPS, CH), jnp.int32),
            pltpu.VMEM((KP, CH, F), jnp.float32),
            pltpu.VMEM_SHARED((N_PAD, F), jnp.float32),
            pltpu.VMEM_SHARED((N_PAD, F), jnp.float32),
            [pltpu.SemaphoreType.DMA] * KP,
            pltpu.SemaphoreType.DMA,
        ],
    )
    def agg(src_hbm, dst_hbm, s_hbm, out_hbm, sidx_v, didx_v, bufs_v, acc_sh,
            s_sh, gsems, ssem):
        cid = lax.axis_index("c")
        sid = lax.axis_index("s")
        wid = sid * NC + cid
        base = sid * ROWS_PER_TILE
        pltpu.sync_copy(s_hbm.at[pl.ds(base, ROWS_PER_TILE)],
                        s_sh.at[pl.ds(base, ROWS_PER_TILE)])
        pltpu.sync_copy(s_hbm.at[pl.ds(base, ROWS_PER_TILE)],
                        acc_sh.at[pl.ds(base, ROWS_PER_TILE)])
        plsc.subcore_barrier()

        def body(p, carry):
            i0 = p * KP
            gathers = [
                pltpu.async_copy(s_sh.at[sidx_v.at[i0 + k]], bufs_v.at[k],
                                 gsems[k])
                for k in range(KP)
            ]
            scatters = []
            for k in range(KP):
                gathers[k].wait()
                scatters.append(
                    pltpu.async_copy(bufs_v.at[k], acc_sh.at[didx_v.at[i0 + k]],
                                     ssem, add=True))
            for k in range(KP):
                scatters[k].wait()
            return carry

        for b in range(NST):
            pltpu.sync_copy(src_hbm.at[wid, pl.ds(b * CPS, CPS)], sidx_v)
            pltpu.sync_copy(dst_hbm.at[wid, pl.ds(b * CPS, CPS)], didx_v)
            lax.fori_loop(0, NIT, body, 0)

        plsc.subcore_barrier()
        pltpu.sync_copy(acc_sh.at[pl.ds(base, ROWS_PER_TILE)],
                        out_hbm.at[cid, pl.ds(base, ROWS_PER_TILE)])

    return agg


@functools.cache
def _make_sc_degree():

    @functools.partial(
        pl.kernel,
        out_type=jax.ShapeDtypeStruct((NC, N_PAD, 8), jnp.float32),
        mesh=plsc.VectorSubcoreMesh(core_axis_name="c", subcore_axis_name="s"),
        compiler_params=pltpu.CompilerParams(use_tc_tiling_on_sc=False),
        scratch_types=[
            pltpu.VMEM((CPW, CH), jnp.int32),
            pltpu.VMEM((CH, 8), jnp.float32),
            pltpu.VMEM_SHARED((N_PAD, 8), jnp.float32),
            pltpu.SemaphoreType.DMA,
        ],
    )
    def _sc_degree(dst_hbm, ones_hbm, out_hbm, didx_v, ones_v, acc_sh, ssem):
        cid = lax.axis_index("c")
        sid = lax.axis_index("s")
        wid = sid * NC + cid
        base = sid * ROWS_PER_TILE
        pltpu.sync_copy(ones_hbm, acc_sh.at[pl.ds(base, ROWS_PER_TILE)])
        pltpu.sync_copy(ones_hbm.at[pl.ds(0, CH)], ones_v)
        pltpu.sync_copy(dst_hbm.at[wid], didx_v)
        plsc.subcore_barrier()

        def body(p, carry):
            i0 = p * K
            scatters = [
                pltpu.async_copy(ones_v, acc_sh.at[didx_v.at[i0 + k]], ssem,
                                 add=True)
                for k in range(K)
            ]
            for k in range(K):
                scatters[k].wait()
            return carry

        lax.fori_loop(0, NITER, body, 0)
        plsc.subcore_barrier()
        pltpu.sync_copy(acc_sh.at[pl.ds(base, ROWS_PER_TILE)],
                        out_hbm.at[cid, pl.ds(base, ROWS_PER_TILE)])

    return _sc_degree


_R = 2048
_GRID = N_PAD // _R


def _tc_mm_body(x_ref, w_ref, u_ref):
    u_ref[...] = jnp.dot(x_ref[...], w_ref[...],
                         preferred_element_type=jnp.float32)


def _tc_mm(x_pad, w1):
    return pl.pallas_call(
        _tc_mm_body,
        grid=(_GRID,),
        in_specs=[
            pl.BlockSpec((_R, 128), lambda i: (i, 0)),
            pl.BlockSpec((128, 64), lambda i: (0, 0)),
        ],
        out_specs=pl.BlockSpec((_R, 64), lambda i: (i, 0)),
        out_shape=jax.ShapeDtypeStruct((N_PAD, 64), jnp.float32),
    )(x_pad, w1)


def _tc_pre_body(parts_ref, u_ref, s_ref, dinv_ref):
    deg = parts_ref[0, :, 0:1] + parts_ref[1, :, 0:1] - 1.0
    dinv = lax.rsqrt(jnp.maximum(deg, 1.0))
    dinv_ref[...] = dinv
    s_ref[...] = u_ref[...] * dinv


def _tc_pre(deg_parts, u):
    return pl.pallas_call(
        _tc_pre_body,
        grid=(_GRID,),
        in_specs=[
            pl.BlockSpec((NC, _R, 8), lambda i: (0, i, 0)),
            pl.BlockSpec((_R, 64), lambda i: (i, 0)),
        ],
        out_specs=[
            pl.BlockSpec((_R, 64), lambda i: (i, 0)),
            pl.BlockSpec((_R, 1), lambda i: (i, 0)),
        ],
        out_shape=[
            jax.ShapeDtypeStruct((N_PAD, 64), jnp.float32),
            jax.ShapeDtypeStruct((N_PAD, 1), jnp.float32),
        ],
    )(deg_parts, u)


def _tc_mid_body(parts_ref, s1_ref, dinv_ref, b1_ref, w2_ref, s2_ref):
    agg = parts_ref[0] + parts_ref[1] - s1_ref[...]
    h = jnp.maximum(agg * dinv_ref[...] + b1_ref[...], 0.0)
    s2_ref[...] = jnp.dot(h, w2_ref[...],
                          preferred_element_type=jnp.float32) * dinv_ref[...]


def _tc_mid(parts1, s1, dinv, b1r, w2p):
    return pl.pallas_call(
        _tc_mid_body,
        grid=(_GRID,),
        in_specs=[
            pl.BlockSpec((NC, _R, 64), lambda i: (0, i, 0)),
            pl.BlockSpec((_R, 64), lambda i: (i, 0)),
            pl.BlockSpec((_R, 1), lambda i: (i, 0)),
            pl.BlockSpec((1, 64), lambda i: (0, 0)),
            pl.BlockSpec((64, 8), lambda i: (0, 0)),
        ],
        out_specs=pl.BlockSpec((_R, 8), lambda i: (i, 0)),
        out_shape=jax.ShapeDtypeStruct((N_PAD, 8), jnp.float32),
    )(parts1, s1, dinv, b1r, w2p)


_PR = 2000


def _tc_post_body(parts_ref, s2_ref, dinv_ref, b2_ref, o_ref):
    agg = parts_ref[0] + parts_ref[1] - s2_ref[...]
    o_ref[...] = (agg * dinv_ref[...] + b2_ref[...])[:, :7]


def _tc_post(parts2, s2, dinv, b2r):
    return pl.pallas_call(
        _tc_post_body,
        grid=(N // _PR,),
        in_specs=[
            pl.BlockSpec((NC, _PR, 8), lambda i: (0, i, 0)),
            pl.BlockSpec((_PR, 8), lambda i: (i, 0)),
            pl.BlockSpec((_PR, 1), lambda i: (i, 0)),
            pl.BlockSpec((1, 8), lambda i: (0, 0)),
        ],
        out_specs=pl.BlockSpec((_PR, 7), lambda i: (i, 0)),
        out_shape=jax.ShapeDtypeStruct((N, 7), jnp.float32),
    )(parts2, s2, dinv, b2r)


def kernel(x, edge_index, W1, b1, W2, b2):
    ei = edge_index.astype(jnp.int32)
    pad_dst = N + (jnp.arange(E_PAD - E, dtype=jnp.int32) % (N_PAD - N))
    src_r = jnp.concatenate(
        [ei[0], jnp.zeros((E_PAD - E,), jnp.int32)]).reshape(NW, CPW, CH)
    dst_r = jnp.concatenate([ei[1], pad_dst]).reshape(NW, CPW, CH)
    x_pad = jnp.pad(x, ((0, N_PAD - N), (0, 0)))
    w2p = jnp.pad(W2, ((0, 0), (0, 1)))
    b1r = b1.reshape(1, 64)
    b2r = jnp.pad(b2, (0, 1)).reshape(1, 8)
    ones = jnp.ones((ROWS_PER_TILE, 8), jnp.float32)

    u = _tc_mm(x_pad, W1)
    deg_parts = _make_sc_degree()(dst_r, ones)
    s1, dinv = _tc_pre(deg_parts, u)
    parts1 = _make_sc_aggregate(64, 4)(src_r, dst_r, s1)
    s2 = _tc_mid(parts1, s1, dinv, b1r, w2p)
    parts2 = _make_sc_aggregate(8, 16)(src_r, dst_r, s2)
    return _tc_post(parts2, s2, dinv, b2r)


# speedup vs baseline: 1.9692x; 1.0017x over previous
import functools

import jax
import jax.numpy as jnp
from jax import lax
from jax.experimental import pallas as pl
from jax.experimental.pallas import tpu as pltpu
from jax.experimental.pallas import tpu_sc as plsc

N = 10000
N_PAD = 10240
E = 320000
CH = 128
NC = 2
NS = 16
NW = NC * NS
ROWS_PER_TILE = N_PAD // NS
K = 8
CPW = 80
E_PAD = NW * CPW * CH
NITER = CPW // K

@functools.cache
def _make_sc_aggregate(F, KP):
    NST = 2 if KP * CH * F * NS + 2 * CPW * CH * NS + 2 * N_PAD * F > 2097151 else 1
    CPS = CPW // NST
    NIT = CPS // KP

    @functools.partial(
        pl.kernel,
        out_type=jax.ShapeDtypeStruct((NC, N_PAD, F), jnp.float32),
        mesh=plsc.VectorSubcoreMesh(core_axis_name="c", subcore_axis_name="s"),
        compiler_params=pltpu.CompilerParams(use_tc_tiling_on_sc=False),
        scratch_types=[
            pltpu.VMEM((CPS, CH), jnp.int32),
            pltpu.VMEM((CPS, CH), jnp.int32),
            pltpu.VMEM((KP, CH, F), jnp.float32),
            pltpu.VMEM_SHARED((N_PAD, F), jnp.float32),
            pltpu.VMEM_SHARED((N_PAD, F), jnp.float32),
            [pltpu.SemaphoreType.DMA] * KP,
            pltpu.SemaphoreType.DMA,
        ],
    )
    def agg(src_hbm, dst_hbm, s_hbm, out_hbm, sidx_v, didx_v, bufs_v, acc_sh,
            s_sh, gsems, ssem):
        cid = lax.axis_index("c")
        sid = lax.axis_index("s")
        wid = sid * NC + cid
        base = sid * ROWS_PER_TILE
        pltpu.sync_copy(s_hbm.at[pl.ds(base, ROWS_PER_TILE)],
                        s_sh.at[pl.ds(base, ROWS_PER_TILE)])
        pltpu.sync_copy(s_hbm.at[pl.ds(base, ROWS_PER_TILE)],
                        acc_sh.at[pl.ds(base, ROWS_PER_TILE)])
        plsc.subcore_barrier()

        def body(p, carry):
            i0 = p * KP
            gathers = [
                pltpu.async_copy(s_sh.at[sidx_v.at[i0 + k]], bufs_v.at[k],
                                 gsems[k])
                for k in range(KP)
            ]
            scatters = []
            for k in range(KP):
                gathers[k].wait()
                scatters.append(
                    pltpu.async_copy(bufs_v.at[k], acc_sh.at[didx_v.at[i0 + k]],
                                     ssem, add=True))
            for k in range(KP):
                scatters[k].wait()
            return carry

        for b in range(NST):
            pltpu.sync_copy(src_hbm.at[wid, pl.ds(b * CPS, CPS)], sidx_v)
            pltpu.sync_copy(dst_hbm.at[wid, pl.ds(b * CPS, CPS)], didx_v)
            lax.fori_loop(0, NIT, body, 0)

        plsc.subcore_barrier()
        pltpu.sync_copy(acc_sh.at[pl.ds(base, ROWS_PER_TILE)],
                        out_hbm.at[cid, pl.ds(base, ROWS_PER_TILE)])

    return agg


@functools.cache
def _make_sc_degree():

    @functools.partial(
        pl.kernel,
        out_type=jax.ShapeDtypeStruct((NC, N_PAD, 8), jnp.float32),
        mesh=plsc.VectorSubcoreMesh(core_axis_name="c", subcore_axis_name="s"),
        compiler_params=pltpu.CompilerParams(use_tc_tiling_on_sc=False),
        scratch_types=[
            pltpu.VMEM((CPW, CH), jnp.int32),
            pltpu.VMEM((CH, 8), jnp.float32),
            pltpu.VMEM_SHARED((N_PAD, 8), jnp.float32),
            pltpu.SemaphoreType.DMA,
        ],
    )
    def _sc_degree(dst_hbm, ones_hbm, out_hbm, didx_v, ones_v, acc_sh, ssem):
        cid = lax.axis_index("c")
        sid = lax.axis_index("s")
        wid = sid * NC + cid
        base = sid * ROWS_PER_TILE
        pltpu.sync_copy(ones_hbm, acc_sh.at[pl.ds(base, ROWS_PER_TILE)])
        pltpu.sync_copy(ones_hbm.at[pl.ds(0, CH)], ones_v)
        pltpu.sync_copy(dst_hbm.at[wid], didx_v)
        plsc.subcore_barrier()

        def body(p, carry):
            i0 = p * K
            scatters = [
                pltpu.async_copy(ones_v, acc_sh.at[didx_v.at[i0 + k]], ssem,
                                 add=True)
                for k in range(K)
            ]
            for k in range(K):
                scatters[k].wait()
            return carry

        lax.fori_loop(0, NITER, body, 0)
        plsc.subcore_barrier()
        pltpu.sync_copy(acc_sh.at[pl.ds(base, ROWS_PER_TILE)],
                        out_hbm.at[cid, pl.ds(base, ROWS_PER_TILE)])

    return _sc_degree


_R = 2048
_GRID = N_PAD // _R


def _tc_mm_body(x_ref, w_ref, u_ref):
    u_ref[...] = jnp.dot(x_ref[...], w_ref[...],
                         preferred_element_type=jnp.float32)


def _tc_mm(x_pad, w1):
    return pl.pallas_call(
        _tc_mm_body,
        grid=(_GRID,),
        in_specs=[
            pl.BlockSpec((_R, 128), lambda i: (i, 0)),
            pl.BlockSpec((128, 64), lambda i: (0, 0)),
        ],
        out_specs=pl.BlockSpec((_R, 64), lambda i: (i, 0)),
        out_shape=jax.ShapeDtypeStruct((N_PAD, 64), jnp.float32),
    )(x_pad, w1)


def _tc_pre_body(parts_ref, u_ref, s_ref, dinv_ref):
    deg = parts_ref[0, :, 0:1] + parts_ref[1, :, 0:1] - 1.0
    dinv = lax.rsqrt(jnp.maximum(deg, 1.0))
    dinv_ref[...] = dinv
    s_ref[...] = u_ref[...] * dinv


def _tc_pre(deg_parts, u):
    return pl.pallas_call(
        _tc_pre_body,
        grid=(_GRID,),
        in_specs=[
            pl.BlockSpec((NC, _R, 8), lambda i: (0, i, 0)),
            pl.BlockSpec((_R, 64), lambda i: (i, 0)),
        ],
        out_specs=[
            pl.BlockSpec((_R, 64), lambda i: (i, 0)),
            pl.BlockSpec((_R, 1), lambda i: (i, 0)),
        ],
        out_shape=[
            jax.ShapeDtypeStruct((N_PAD, 64), jnp.float32),
            jax.ShapeDtypeStruct((N_PAD, 1), jnp.float32),
        ],
    )(deg_parts, u)


def _tc_mid_body(parts_ref, s1_ref, dinv_ref, b1_ref, w2_ref, s2_ref):
    agg = parts_ref[0] + parts_ref[1] - s1_ref[...]
    h = jnp.maximum(agg * dinv_ref[...] + b1_ref[...], 0.0)
    s2_ref[...] = jnp.dot(h, w2_ref[...],
                          preferred_element_type=jnp.float32) * dinv_ref[...]


def _tc_mid(parts1, s1, dinv, b1r, w2p):
    return pl.pallas_call(
        _tc_mid_body,
        grid=(_GRID,),
        in_specs=[
            pl.BlockSpec((NC, _R, 64), lambda i: (0, i, 0)),
            pl.BlockSpec((_R, 64), lambda i: (i, 0)),
            pl.BlockSpec((_R, 1), lambda i: (i, 0)),
            pl.BlockSpec((1, 64), lambda i: (0, 0)),
            pl.BlockSpec((64, 8), lambda i: (0, 0)),
        ],
        out_specs=pl.BlockSpec((_R, 8), lambda i: (i, 0)),
        out_shape=jax.ShapeDtypeStruct((N_PAD, 8), jnp.float32),
    )(parts1, s1, dinv, b1r, w2p)


_PR = 2000


def _tc_post_body(parts_ref, s2_ref, dinv_ref, b2_ref, o_ref):
    agg = parts_ref[0] + parts_ref[1] - s2_ref[...]
    o_ref[...] = (agg * dinv_ref[...] + b2_ref[...])[:, :7]


def _tc_post(parts2, s2, dinv, b2r):
    return pl.pallas_call(
        _tc_post_body,
        grid=(N // _PR,),
        in_specs=[
            pl.BlockSpec((NC, _PR, 8), lambda i: (0, i, 0)),
            pl.BlockSpec((_PR, 8), lambda i: (i, 0)),
            pl.BlockSpec((_PR, 1), lambda i: (i, 0)),
            pl.BlockSpec((1, 8), lambda i: (0, 0)),
        ],
        out_specs=pl.BlockSpec((_PR, 7), lambda i: (i, 0)),
        out_shape=jax.ShapeDtypeStruct((N, 7), jnp.float32),
    )(parts2, s2, dinv, b2r)


def kernel(x, edge_index, W1, b1, W2, b2):
    ei = edge_index.astype(jnp.int32)
    pad_dst = N + (jnp.arange(E_PAD - E, dtype=jnp.int32) % (N_PAD - N))
    src_r = jnp.concatenate(
        [ei[0], jnp.zeros((E_PAD - E,), jnp.int32)]).reshape(NW, CPW, CH)
    dst_r = jnp.concatenate([ei[1], pad_dst]).reshape(NW, CPW, CH)
    x_pad = jnp.pad(x, ((0, N_PAD - N), (0, 0)))
    w2p = jnp.pad(W2, ((0, 0), (0, 1)))
    b1r = b1.reshape(1, 64)
    b2r = jnp.pad(b2, (0, 1)).reshape(1, 8)
    ones = jnp.ones((ROWS_PER_TILE, 8), jnp.float32)

    u = _tc_mm(x_pad, W1)
    deg_parts = _make_sc_degree()(dst_r, ones)
    s1, dinv = _tc_pre(deg_parts, u)
    parts1 = _make_sc_aggregate(64, 4)(src_r, dst_r, s1)
    s2 = _tc_mid(parts1, s1, dinv, b1r, w2p)
    parts2 = _make_sc_aggregate(8, 16)(src_r, dst_r, s2)
    return _tc_post(parts2, s2, dinv, b2r)
